# baseline (speedup 1.0000x reference)
import functools

import jax
import jax.numpy as jnp
from jax import lax
from jax.experimental import pallas as pl
from jax.experimental.pallas import tpu as pltpu
from jax.experimental.pallas import tpu_sc as plsc

NC = 2
NS = 16
NW = NC * NS
LANES = 16


@functools.lru_cache(maxsize=None)
def _make_pool(B, L, V, E, interpret=False):
    assert B % NW == 0
    bpw = B // NW
    Q = 16
    RQ = bpw // Q
    IB = 50
    assert L % IB == 0
    nblk = L // IB

    mesh = plsc.VectorSubcoreMesh(
        core_axis_name="c", subcore_axis_name="s", num_cores=NC, num_subcores=NS)

    @functools.partial(
        pl.kernel,
        out_type=jax.ShapeDtypeStruct((B, E), jnp.float32),
        mesh=mesh,
        scratch_types=[
            pltpu.VMEM((2, IB, bpw), jnp.int32),
            pltpu.VMEM((bpw, E), jnp.float32),
        ] + [pltpu.SemaphoreType.DMA] * Q + [
            pltpu.SemaphoreType.DMA,
            pltpu.SemaphoreType.DMA,
        ],
        compiler_params=pltpu.CompilerParams(use_tc_tiling_on_sc=False),
        interpret=interpret,
    )
    def pool(xt_hbm, emb_hbm, out_hbm, idxblk, outbuf, *sems):
        wid = lax.axis_index("s") * NC + lax.axis_index("c")
        base = wid * bpw
        gsems = sems[:Q]
        isems = sems[Q:Q + 2]

        def idx_load(b):
            return pltpu.make_async_copy(
                xt_hbm.at[pl.ds(b * IB, IB), pl.ds(base, bpw)],
                idxblk.at[b % 2], isems[b % 2])

        def issue(buf, l, j, add):
            src = emb_hbm.at[idxblk.at[buf, l, pl.ds(j * RQ, RQ)]]
            pltpu.async_copy(src, outbuf.at[pl.ds(j * RQ, RQ)], gsems[j], add=add)

        def wait_chain(buf, l, j):
            src = emb_hbm.at[idxblk.at[buf, l, pl.ds(j * RQ, RQ)]]
            pltpu.make_async_copy(
                src, outbuf.at[pl.ds(j * RQ, RQ)], gsems[j]).wait()

        idx_load(0).start()
        idx_load(0).wait()

        for b in range(nblk):
            buf = b % 2
            if b + 1 < nblk:
                idx_load(b + 1).start()
            if b == 0:
                for j in range(Q):
                    issue(buf, 0, j, add=False)

                @pl.loop(1, IB)
                def _ls(l):
                    for j in range(Q):
                        wait_chain(buf, l, j)
                        issue(buf, l, j, add=True)
            else:
                idx_load(b).wait()

                @pl.loop(0, IB)
                def _ls(l, buf=buf):
                    for j in range(Q):
                        wait_chain(buf, l, j)
                        issue(buf, l, j, add=True)

        for j in range(Q):
            wait_chain((nblk - 1) % 2, IB - 1, j)

        pltpu.sync_copy(outbuf, out_hbm.at[pl.ds(base, bpw)])

    return pool


@functools.lru_cache(maxsize=None)
def _make_mlp(B, E, H2, H, N, inv_l, interpret=False):
    BM = min(B, 2048)
    assert B % BM == 0

    def body(s_ref, w1_ref, b1_ref, w2_ref, b2_ref, w3_ref, b3_ref, o_ref):
        p = s_ref[...] * inv_l
        h = jnp.dot(p, w1_ref[...], preferred_element_type=jnp.float32)
        h = jnp.maximum(h + b1_ref[...], 0.0)
        h = jnp.dot(h, w2_ref[...], preferred_element_type=jnp.float32)
        h = jnp.maximum(h + b2_ref[...], 0.0)
        o = jnp.dot(h, w3_ref[...], preferred_element_type=jnp.float32)
        o_ref[...] = o + b3_ref[...]

    zero = lambda i: (0, 0)
    return pl.pallas_call(
        body,
        grid=(B // BM,),
        in_specs=[
            pl.BlockSpec((BM, E), lambda i: (i, 0)),
            pl.BlockSpec((E, H2), zero),
            pl.BlockSpec((1, H2), zero),
            pl.BlockSpec((H2, H), zero),
            pl.BlockSpec((1, H), zero),
            pl.BlockSpec((H, N), zero),
            pl.BlockSpec((1, N), zero),
        ],
        out_specs=pl.BlockSpec((BM, N), lambda i: (i, 0)),
        out_shape=jax.ShapeDtypeStruct((B, N), jnp.float32),
        interpret=interpret,
    )


def _run(x, emb, W1, b1, W2, b2, W3, b3, interpret=False):
    B, L = x.shape
    V, E = emb.shape
    H2 = W1.shape[1]
    H = W2.shape[1]
    N = W3.shape[1]
    xt = x.astype(jnp.int32).T
    sums = _make_pool(B, L, V, E, interpret)(xt, emb)
    mlp = _make_mlp(B, E, H2, H, N, 1.0 / L, interpret)
    return mlp(sums, W1, b1.reshape(1, -1), W2, b2.reshape(1, -1),
               W3, b3.reshape(1, -1))


def kernel(x, emb, W1, b1, W2, b2, W3, b3):
    return _run(x, emb, W1, b1, W2, b2, W3, b3)


# speedup vs baseline: 1.1049x; 1.1049x over previous
import functools

import jax
import jax.numpy as jnp
from jax import lax
from jax.experimental import pallas as pl
from jax.experimental.pallas import tpu as pltpu
from jax.experimental.pallas import tpu_sc as plsc

NC = 2
NS = 16
NW = NC * NS
LANES = 16


@functools.lru_cache(maxsize=None)
def _make_pool(B, L, V, E, interpret=False):
    assert B % NW == 0
    bpw = B // NW
    ecols = E // LANES

    mesh = plsc.VectorSubcoreMesh(
        core_axis_name="c", subcore_axis_name="s", num_cores=NC, num_subcores=NS)

    IBLK = 32
    NBUF = 4
    assert bpw % NBUF == 0 and bpw % IBLK == 0

    @functools.partial(
        pl.kernel,
        out_type=jax.ShapeDtypeStruct((B, E), jnp.float32),
        mesh=mesh,
        scratch_types=[
            pltpu.VMEM((2, IBLK, L), jnp.int32),
            pltpu.VMEM((NBUF, L, E), jnp.float32),
            pltpu.VMEM((bpw, E), jnp.float32),
            pltpu.SemaphoreType.DMA,
            pltpu.SemaphoreType.DMA,
            pltpu.SemaphoreType.DMA,
            pltpu.SemaphoreType.DMA,
        ],
        compiler_params=pltpu.CompilerParams(use_tc_tiling_on_sc=False),
        interpret=interpret,
    )
    def pool(x_hbm, emb_hbm, out_hbm, idxblk, rows_v, outbuf, *sems):
        wid = lax.axis_index("s") * NC + lax.axis_index("c")
        base = wid * bpw

        def load_iblk(r):
            blk = r // IBLK
            pltpu.sync_copy(
                x_hbm.at[pl.ds(base + blk * IBLK, IBLK)], idxblk.at[blk % 2])

        def idx_view(r):
            return idxblk.at[(r // IBLK) % 2, r % IBLK]

        def start_row(r, b):
            pltpu.async_copy(emb_hbm.at[idx_view(r)], rows_v.at[b], sems[b])

        def wait_row(r, b):
            pltpu.make_async_copy(
                emb_hbm.at[idx_view(r)], rows_v.at[b], sems[b]).wait()

        load_iblk(0)
        for j in range(NBUF - 1):
            start_row(j, j)

        @pl.loop(0, bpw, step=NBUF)
        def _rows(r):
            for j in range(NBUF):
                rr = r + j
                nxt = rr + (NBUF - 1)

                @pl.when(jnp.logical_and(nxt % IBLK == 0, nxt < bpw))
                def _():
                    load_iblk(nxt)

                @pl.when(nxt < bpw)
                def _():
                    start_row(nxt, (j + NBUF - 1) % NBUF)

                wait_row(rr, j)

                zeros = tuple(jnp.zeros((LANES,), jnp.float32) for _ in range(ecols))

                @pl.loop(0, L, init_carry=zeros, unroll=8)
                def _red(k, carry):
                    return tuple(
                        carry[c] + rows_v[j, k, pl.ds(c * LANES, LANES)]
                        for c in range(ecols))

                acc = _red
                for c in range(ecols):
                    outbuf[rr, pl.ds(c * LANES, LANES)] = acc[c]

        pltpu.sync_copy(outbuf, out_hbm.at[pl.ds(base, bpw)])

    return pool


@functools.lru_cache(maxsize=None)
def _make_mlp(B, E, H2, H, N, inv_l, interpret=False):
    BM = min(B, 2048)
    assert B % BM == 0

    def body(s_ref, w1_ref, b1_ref, w2_ref, b2_ref, w3_ref, b3_ref, o_ref):
        p = s_ref[...] * inv_l
        h = jnp.dot(p, w1_ref[...], preferred_element_type=jnp.float32)
        h = jnp.maximum(h + b1_ref[...], 0.0)
        h = jnp.dot(h, w2_ref[...], preferred_element_type=jnp.float32)
        h = jnp.maximum(h + b2_ref[...], 0.0)
        o = jnp.dot(h, w3_ref[...], preferred_element_type=jnp.float32)
        o_ref[...] = o + b3_ref[...]

    zero = lambda i: (0, 0)
    return pl.pallas_call(
        body,
        grid=(B // BM,),
        in_specs=[
            pl.BlockSpec((BM, E), lambda i: (i, 0)),
            pl.BlockSpec((E, H2), zero),
            pl.BlockSpec((1, H2), zero),
            pl.BlockSpec((H2, H), zero),
            pl.BlockSpec((1, H), zero),
            pl.BlockSpec((H, N), zero),
            pl.BlockSpec((1, N), zero),
        ],
        out_specs=pl.BlockSpec((BM, N), lambda i: (i, 0)),
        out_shape=jax.ShapeDtypeStruct((B, N), jnp.float32),
        interpret=interpret,
    )


def _run(x, emb, W1, b1, W2, b2, W3, b3, interpret=False):
    B, L = x.shape
    V, E = emb.shape
    H2 = W1.shape[1]
    H = W2.shape[1]
    N = W3.shape[1]
    sums = _make_pool(B, L, V, E, interpret)(x.astype(jnp.int32), emb)
    mlp = _make_mlp(B, E, H2, H, N, 1.0 / L, interpret)
    return mlp(sums, W1, b1.reshape(1, -1), W2, b2.reshape(1, -1),
               W3, b3.reshape(1, -1))


def kernel(x, emb, W1, b1, W2, b2, W3, b3):
    return _run(x, emb, W1, b1, W2, b2, W3, b3)
